# baseline (speedup 1.0000x reference)
import functools

import jax
import jax.numpy as jnp
from jax.experimental import pallas as pl

_B = 4096
_D = 64
_H = _B
_N = 2 * _B
_INV_T = 2.0
_G = (_N * (_N - 2) - 2 * _B) // _N
_BIG = 2 ** 30
_R1 = 1024
_R2 = 1024


def _topk_body(hi_ref, hj_ref, vals_ref, idxs_ref):
    i0 = pl.program_id(0) * _R1
    a = hi_ref[pl.ds(i0, _R1), :]
    b = hj_ref[pl.ds(i0, _R1), :]
    dn = (((1,), (1,)), ((), ()))
    x1 = jax.lax.dot_general(a, hj_ref[:], dn,
                             preferred_element_type=jnp.float32) * _INV_T
    x2 = jax.lax.dot_general(b, hi_ref[:], dn,
                             preferred_element_type=jnp.float32) * _INV_T
    col1 = jax.lax.broadcasted_iota(jnp.int32, (_R1, _H), 1)
    col2 = col1 + _H
    neg_inf = jnp.float32(-jnp.inf)
    vs, ids = [], []
    for _ in range(4):
        m = jnp.maximum(jnp.max(x1, axis=1), jnp.max(x2, axis=1))
        am1 = jnp.min(jnp.where(x1 == m[:, None], col1, _BIG), axis=1)
        am2 = jnp.min(jnp.where(x2 == m[:, None], col2, _BIG), axis=1)
        am = jnp.minimum(am1, am2)
        vs.append(m[:, None])
        ids.append(am[:, None])
        x1 = jnp.where(col1 == am[:, None], neg_inf, x1)
        x2 = jnp.where(col2 == am[:, None], neg_inf, x2)
    vals_ref[...] = jnp.concatenate(vs, axis=1)
    idxs_ref[...] = jnp.concatenate(ids, axis=1)


def _lse_body(h_ref, e_ref, c_ref, out_ref):
    i0 = pl.program_id(0) * _R2
    a = h_ref[pl.ds(i0, _R2), :]
    dn = (((1,), (1,)), ((), ()))
    x = jax.lax.dot_general(a, h_ref[:], dn,
                            preferred_element_type=jnp.float32) * _INV_T
    col = jax.lax.broadcasted_iota(jnp.int32, (_R2, _N), 1)
    e = e_ref[...]
    excl = ((col == e[:, 0:1]) | (col == e[:, 1:2])
            | (col == e[:, 2:3]) | (col == e[:, 3:4]))
    xn = jnp.where(excl, jnp.float32(-jnp.inf), x)
    m = jnp.max(xn, axis=1)
    ex = jnp.exp(xn - m[:, None])
    c1 = c_ref[:, 0:1]
    c2 = c_ref[:, 1:2]
    zero = jnp.float32(0.0)
    u0 = jnp.sum(jnp.where(col < c1, ex, zero), axis=1)
    u1 = jnp.sum(jnp.where(col < c2, ex, zero), axis=1)
    ut = jnp.sum(ex, axis=1)
    out_ref[...] = jnp.concatenate(
        [m[:, None], u0[:, None], (u1 - u0)[:, None], (ut - u1)[:, None]],
        axis=1)


@functools.partial(jax.jit)
def kernel(h_i, h_j):
    full_i = pl.BlockSpec((_H, _D), lambda i: (0, 0))
    vals, idxs = pl.pallas_call(
        _topk_body,
        grid=(_H // _R1,),
        in_specs=[full_i, full_i],
        out_specs=[pl.BlockSpec((_R1, 4), lambda i: (i, 0)),
                   pl.BlockSpec((_R1, 4), lambda i: (i, 0))],
        out_shape=[jax.ShapeDtypeStruct((_H, 4), jnp.float32),
                   jax.ShapeDtypeStruct((_H, 4), jnp.int32)],
    )(h_i, h_j)

    i = jnp.arange(_H, dtype=jnp.int32)
    valid = (idxs != i[:, None]) & (idxs != (_H + i)[:, None])
    taken = valid & (jnp.cumsum(valid.astype(jnp.int32), axis=1) <= 2)
    ge = idxs >= _H

    lo_mask = taken & ~ge
    lo_key = jnp.where(lo_mask, idxs + _H, _BIG)
    lo_ord = jnp.argsort(lo_key, axis=1)
    lo_cols = jnp.take_along_axis(lo_key, lo_ord, axis=1)[:, :2]
    lo_vals = jnp.take_along_axis(jnp.where(lo_mask, vals, 0.0),
                                  lo_ord, axis=1)[:, :2]
    n_lo = jnp.sum(lo_mask, axis=1).astype(jnp.int32)

    hi_mask = taken & ge
    hi_key = jnp.where(hi_mask, idxs - _H, _BIG)
    hi_ord = jnp.argsort(hi_key, axis=1)
    hi_cols = jnp.take_along_axis(hi_key, hi_ord, axis=1)[:, :2]
    hi_vals = jnp.take_along_axis(jnp.where(hi_mask, vals, 0.0),
                                  hi_ord, axis=1)[:, :2]
    n_hi = jnp.sum(hi_mask, axis=1).astype(jnp.int32)

    extra_cols = jnp.concatenate([lo_cols, hi_cols], axis=0)
    extra_vals = jnp.concatenate([lo_vals, hi_vals], axis=0)
    n_extra = jnp.concatenate([n_lo, n_hi], axis=0)

    r = jnp.arange(_N, dtype=jnp.int32)
    i_of_r = jnp.where(r < _H, r, r - _H)
    d_r = (_N - 2) - n_extra
    c_r = 1 + n_extra
    S = jnp.concatenate([jnp.zeros((1,), jnp.int32), jnp.cumsum(d_r)])
    P = jnp.concatenate([jnp.zeros((1,), jnp.int32), jnp.cumsum(c_r)])

    base_col = jnp.where(r < _H, r + _H, r - _H)
    diag_m = jnp.sum(h_i * h_j, axis=1) * _INV_T
    base_val = jnp.concatenate([diag_m, diag_m])

    b_lt0 = base_col < extra_cols[:, 0]
    b_lt1 = base_col < extra_cols[:, 1]
    pv0 = jnp.where(b_lt0, base_val, extra_vals[:, 0])
    pv1 = jnp.where(b_lt0, extra_vals[:, 0],
                    jnp.where(b_lt1, base_val, extra_vals[:, 1]))
    pv2 = jnp.where(b_lt1, extra_vals[:, 1], base_val)
    pos3 = jnp.stack([pv0, pv1, pv2], axis=1)
    slot = P[:_N, None] + jnp.arange(3, dtype=jnp.int32)[None, :]
    slot = jnp.where(jnp.arange(3)[None, :] < c_r[:, None], slot, 2 * _N)
    pos_flat = jnp.zeros((2 * _N + 1,), jnp.float32).at[
        slot.ravel()].set(pos3.ravel())[: 2 * _N]

    e_mat = jnp.concatenate(
        [i_of_r[:, None], (i_of_r + _H)[:, None], extra_cols], axis=1)
    b1 = (S[:_N] // _G + 1) * _G - S[:_N]
    k1 = jnp.where(b1 < d_r, b1, d_r)
    b2 = b1 + _G
    k2 = jnp.where(b2 < d_r, b2, d_r)

    e_sorted = jnp.sort(e_mat, axis=1)

    def _rank_to_col(k):
        c = k
        for t in range(4):
            c = c + (e_sorted[:, t] <= c).astype(jnp.int32)
        return c

    k_mat = jnp.stack([_rank_to_col(k1), _rank_to_col(k2)],
                      axis=1).astype(jnp.int32)

    h = jnp.concatenate([h_i, h_j], axis=0)
    full_h = pl.BlockSpec((_N, _D), lambda i: (0, 0))
    out = pl.pallas_call(
        _lse_body,
        grid=(_N // _R2,),
        in_specs=[full_h,
                  pl.BlockSpec((_R2, 4), lambda i: (i, 0)),
                  pl.BlockSpec((_R2, 2), lambda i: (i, 0))],
        out_specs=pl.BlockSpec((_R2, 4), lambda i: (i, 0)),
        out_shape=jax.ShapeDtypeStruct((_N, 4), jnp.float32),
    )(h, e_mat.astype(jnp.int32), k_mat)

    q = (S[:_N] + _G - 1) // _G
    marks = jnp.full((_N + 1,), -1, jnp.int32).at[q].max(r)
    a_row = jax.lax.cummax(marks)[:_N]

    out_next = jnp.concatenate([out[1:], out[-1:]], axis=0)
    packed_f = jnp.concatenate([out, out_next[:, :2]], axis=1)
    packed_i = jnp.concatenate(
        [S[:_N, None], S[1:, None], k1[:, None], k2[:, None]],
        axis=1).astype(jnp.int32)
    gf = jnp.take(packed_f, a_row, axis=0)
    gi = jnp.take(packed_i, a_row, axis=0)

    g = jnp.arange(_N, dtype=jnp.int32)
    start = g * _G
    kstart = start - gi[:, 0]
    p_idx = ((kstart >= gi[:, 2]).astype(jnp.int32)
             + (kstart >= gi[:, 3]).astype(jnp.int32))
    spill = start + _G - gi[:, 1]
    has_b = spill > 0
    m_a = gf[:, 0]
    s_a = jnp.where(p_idx == 0, gf[:, 1],
                    jnp.where(p_idx == 1, gf[:, 2], gf[:, 3]))
    m_b = jnp.where(has_b, gf[:, 4], -jnp.inf)
    s_b = jnp.where(has_b, gf[:, 5], 0.0)
    pos_pair = pos_flat.reshape(_N, 2)
    p0 = pos_pair[:, 0]
    p1 = pos_pair[:, 1]
    mx = jnp.maximum(jnp.maximum(m_a, m_b), jnp.maximum(p0, p1))
    tot = (s_a * jnp.exp(m_a - mx) + s_b * jnp.exp(m_b - mx)
           + jnp.exp(p0 - mx) + jnp.exp(p1 - mx))
    lse = mx + jnp.log(tot + jnp.float32(1e-38))
    return (jnp.sum(lse) - jnp.sum(p0)) / _N


# speedup vs baseline: 1.0849x; 1.0849x over previous
import functools

import jax
import jax.numpy as jnp
from jax.experimental import pallas as pl

_B = 4096
_D = 64
_H = _B
_N = 2 * _B
_INV_T = 2.0
_G = (_N * (_N - 2) - 2 * _B) // _N
_BIG = 2 ** 30
_R1 = 512
_R2 = 1024


def _topk_body(hi_ref, hj_ref, vals_ref, idxs_ref):
    i0 = pl.program_id(0) * _R1
    a = hi_ref[pl.ds(i0, _R1), :]
    b = hj_ref[pl.ds(i0, _R1), :]
    dn = (((1,), (1,)), ((), ()))
    x1 = jax.lax.dot_general(a, hj_ref[:], dn,
                             preferred_element_type=jnp.float32) * _INV_T
    x2 = jax.lax.dot_general(b, hi_ref[:], dn,
                             preferred_element_type=jnp.float32) * _INV_T
    col1 = jax.lax.broadcasted_iota(jnp.int32, (_R1, _H), 1)
    col2 = col1 + _H
    neg_inf = jnp.float32(-jnp.inf)
    vs, ids = [], []
    for _ in range(4):
        m = jnp.maximum(jnp.max(x1, axis=1), jnp.max(x2, axis=1))
        am1 = jnp.min(jnp.where(x1 == m[:, None], col1, _BIG), axis=1)
        am2 = jnp.min(jnp.where(x2 == m[:, None], col2, _BIG), axis=1)
        am = jnp.minimum(am1, am2)
        vs.append(m[:, None])
        ids.append(am[:, None])
        x1 = jnp.where(col1 == am[:, None], neg_inf, x1)
        x2 = jnp.where(col2 == am[:, None], neg_inf, x2)
    vals_ref[...] = jnp.concatenate(vs, axis=1)
    idxs_ref[...] = jnp.concatenate(ids, axis=1)


def _lse_body(h_ref, e_ref, c_ref, out_ref):
    i0 = pl.program_id(0) * _R2
    a = h_ref[pl.ds(i0, _R2), :]
    dn = (((1,), (1,)), ((), ()))
    x = jax.lax.dot_general(a, h_ref[:], dn,
                            preferred_element_type=jnp.float32) * _INV_T
    col = jax.lax.broadcasted_iota(jnp.int32, (_R2, _N), 1)
    e = e_ref[...]
    excl = ((col == e[:, 0:1]) | (col == e[:, 1:2])
            | (col == e[:, 2:3]) | (col == e[:, 3:4]))
    xn = jnp.where(excl, jnp.float32(-jnp.inf), x)
    m = jnp.max(xn, axis=1)
    ex = jnp.exp(xn - m[:, None])
    c1 = c_ref[:, 0:1]
    c2 = c_ref[:, 1:2]
    zero = jnp.float32(0.0)
    u0 = jnp.sum(jnp.where(col < c1, ex, zero), axis=1)
    u1 = jnp.sum(jnp.where(col < c2, ex, zero), axis=1)
    ut = jnp.sum(ex, axis=1)
    out_ref[...] = jnp.concatenate(
        [m[:, None], u0[:, None], (u1 - u0)[:, None], (ut - u1)[:, None]],
        axis=1)


@functools.partial(jax.jit)
def kernel(h_i, h_j):
    full_i = pl.BlockSpec((_H, _D), lambda i: (0, 0))
    vals, idxs = pl.pallas_call(
        _topk_body,
        grid=(_H // _R1,),
        in_specs=[full_i, full_i],
        out_specs=[pl.BlockSpec((_R1, 4), lambda i: (i, 0)),
                   pl.BlockSpec((_R1, 4), lambda i: (i, 0))],
        out_shape=[jax.ShapeDtypeStruct((_H, 4), jnp.float32),
                   jax.ShapeDtypeStruct((_H, 4), jnp.int32)],
    )(h_i, h_j)

    i = jnp.arange(_H, dtype=jnp.int32)
    valid = (idxs != i[:, None]) & (idxs != (_H + i)[:, None])
    taken = valid & (jnp.cumsum(valid.astype(jnp.int32), axis=1) <= 2)
    ge = idxs >= _H

    lo_mask = taken & ~ge
    lo_key = jnp.where(lo_mask, idxs + _H, _BIG)
    lo_ord = jnp.argsort(lo_key, axis=1)
    lo_cols = jnp.take_along_axis(lo_key, lo_ord, axis=1)[:, :2]
    lo_vals = jnp.take_along_axis(jnp.where(lo_mask, vals, 0.0),
                                  lo_ord, axis=1)[:, :2]
    n_lo = jnp.sum(lo_mask, axis=1).astype(jnp.int32)

    hi_mask = taken & ge
    hi_key = jnp.where(hi_mask, idxs - _H, _BIG)
    hi_ord = jnp.argsort(hi_key, axis=1)
    hi_cols = jnp.take_along_axis(hi_key, hi_ord, axis=1)[:, :2]
    hi_vals = jnp.take_along_axis(jnp.where(hi_mask, vals, 0.0),
                                  hi_ord, axis=1)[:, :2]
    n_hi = jnp.sum(hi_mask, axis=1).astype(jnp.int32)

    extra_cols = jnp.concatenate([lo_cols, hi_cols], axis=0)
    extra_vals = jnp.concatenate([lo_vals, hi_vals], axis=0)
    n_extra = jnp.concatenate([n_lo, n_hi], axis=0)

    r = jnp.arange(_N, dtype=jnp.int32)
    i_of_r = jnp.where(r < _H, r, r - _H)
    d_r = (_N - 2) - n_extra
    c_r = 1 + n_extra
    S = jnp.concatenate([jnp.zeros((1,), jnp.int32), jnp.cumsum(d_r)])
    P = jnp.concatenate([jnp.zeros((1,), jnp.int32), jnp.cumsum(c_r)])

    base_col = jnp.where(r < _H, r + _H, r - _H)
    diag_m = jnp.sum(h_i * h_j, axis=1) * _INV_T
    base_val = jnp.concatenate([diag_m, diag_m])

    b_lt0 = base_col < extra_cols[:, 0]
    b_lt1 = base_col < extra_cols[:, 1]
    pv0 = jnp.where(b_lt0, base_val, extra_vals[:, 0])
    pv1 = jnp.where(b_lt0, extra_vals[:, 0],
                    jnp.where(b_lt1, base_val, extra_vals[:, 1]))
    pv2 = jnp.where(b_lt1, extra_vals[:, 1], base_val)
    pos3 = jnp.stack([pv0, pv1, pv2], axis=1)
    slot = P[:_N, None] + jnp.arange(3, dtype=jnp.int32)[None, :]
    slot = jnp.where(jnp.arange(3)[None, :] < c_r[:, None], slot, 2 * _N)
    pos_flat = jnp.zeros((2 * _N + 1,), jnp.float32).at[
        slot.ravel()].set(pos3.ravel())[: 2 * _N]

    e_mat = jnp.concatenate(
        [i_of_r[:, None], (i_of_r + _H)[:, None], extra_cols], axis=1)
    b1 = (S[:_N] // _G + 1) * _G - S[:_N]
    k1 = jnp.where(b1 < d_r, b1, d_r)
    b2 = b1 + _G
    k2 = jnp.where(b2 < d_r, b2, d_r)

    e_sorted = jnp.sort(e_mat, axis=1)

    def _rank_to_col(k):
        c = k
        for t in range(4):
            c = c + (e_sorted[:, t] <= c).astype(jnp.int32)
        return c

    k_mat = jnp.stack([_rank_to_col(k1), _rank_to_col(k2)],
                      axis=1).astype(jnp.int32)

    h = jnp.concatenate([h_i, h_j], axis=0)
    full_h = pl.BlockSpec((_N, _D), lambda i: (0, 0))
    out = pl.pallas_call(
        _lse_body,
        grid=(_N // _R2,),
        in_specs=[full_h,
                  pl.BlockSpec((_R2, 4), lambda i: (i, 0)),
                  pl.BlockSpec((_R2, 2), lambda i: (i, 0))],
        out_specs=pl.BlockSpec((_R2, 4), lambda i: (i, 0)),
        out_shape=jax.ShapeDtypeStruct((_N, 4), jnp.float32),
    )(h, e_mat.astype(jnp.int32), k_mat)

    q = (S[:_N] + _G - 1) // _G
    marks = jnp.full((_N + 1,), -1, jnp.int32).at[q].max(r)
    a_row = jax.lax.cummax(marks)[:_N]

    out_next = jnp.concatenate([out[1:], out[-1:]], axis=0)
    packed_f = jnp.concatenate([out, out_next[:, :2]], axis=1)
    packed_i = jnp.concatenate(
        [S[:_N, None], S[1:, None], k1[:, None], k2[:, None]],
        axis=1).astype(jnp.int32)
    gf = jnp.take(packed_f, a_row, axis=0)
    gi = jnp.take(packed_i, a_row, axis=0)

    g = jnp.arange(_N, dtype=jnp.int32)
    start = g * _G
    kstart = start - gi[:, 0]
    p_idx = ((kstart >= gi[:, 2]).astype(jnp.int32)
             + (kstart >= gi[:, 3]).astype(jnp.int32))
    spill = start + _G - gi[:, 1]
    has_b = spill > 0
    m_a = gf[:, 0]
    s_a = jnp.where(p_idx == 0, gf[:, 1],
                    jnp.where(p_idx == 1, gf[:, 2], gf[:, 3]))
    m_b = jnp.where(has_b, gf[:, 4], -jnp.inf)
    s_b = jnp.where(has_b, gf[:, 5], 0.0)
    pos_pair = pos_flat.reshape(_N, 2)
    p0 = pos_pair[:, 0]
    p1 = pos_pair[:, 1]
    mx = jnp.maximum(jnp.maximum(m_a, m_b), jnp.maximum(p0, p1))
    tot = (s_a * jnp.exp(m_a - mx) + s_b * jnp.exp(m_b - mx)
           + jnp.exp(p0 - mx) + jnp.exp(p1 - mx))
    lse = mx + jnp.log(tot + jnp.float32(1e-38))
    return (jnp.sum(lse) - jnp.sum(p0)) / _N
